# baseline (speedup 1.0000x reference)
import functools

import jax
import jax.numpy as jnp
from jax import lax
from jax.experimental import pallas as pl
from jax.experimental.pallas import tpu as pltpu
from jax.experimental.pallas import tpu_sc as plsc

B = 4096
H = 50
D = 64
N = B * H
NC = 2
NS = 16
NW = NC * NS
PER_W = N // NW
R = 400
NBUF = 2

SIZES = [R] * 15 + [208, 96, 48, 32, 16]
OFFS = [sum(SIZES[:i]) for i in range(len(SIZES))]
NCHUNK = len(SIZES)
assert sum(SIZES) == PER_W
UNIFORM = 15


def _sc_body(sid_hbm, fid_hbm, stab_hbm, ftab_hbm, out_hbm,
             sidx_v, fidx_v, s0, s1, f0, f1, out_v,
             sem_s0, sem_s1, sem_f0, sem_f1):
    wid = lax.axis_index("s") * NC + lax.axis_index("c")

    sbufs = (s0, s1)
    fbufs = (f0, f1)
    ssems = (sem_s0, sem_s1)
    fsems = (sem_f0, sem_f1)

    pltpu.sync_copy(sid_hbm.at[wid], sidx_v)
    pltpu.sync_copy(fid_hbm.at[wid], fidx_v)

    def start_dyn(off, k):
        pltpu.async_copy(stab_hbm.at[sidx_v.at[pl.ds(off, R)]],
                         sbufs[k], ssems[k])
        pltpu.async_copy(ftab_hbm.at[fidx_v.at[pl.ds(off, R)]],
                         fbufs[k], fsems[k])

    def wait_dyn(off, k):
        pltpu.make_async_copy(stab_hbm.at[sidx_v.at[pl.ds(off, R)]],
                              sbufs[k], ssems[k]).wait()
        pltpu.make_async_copy(ftab_hbm.at[fidx_v.at[pl.ds(off, R)]],
                              fbufs[k], fsems[k]).wait()

    def _dst(buf, sz):
        return buf if sz == R else buf.at[pl.ds(0, sz)]

    def start_tail(c, k):
        off, sz = OFFS[c], SIZES[c]
        pltpu.async_copy(stab_hbm.at[sidx_v.at[pl.ds(off, sz)]],
                         _dst(sbufs[k], sz), ssems[k])
        pltpu.async_copy(ftab_hbm.at[fidx_v.at[pl.ds(off, sz)]],
                         _dst(fbufs[k], sz), fsems[k])

    def wait_tail(c, k):
        off, sz = OFFS[c], SIZES[c]
        pltpu.make_async_copy(stab_hbm.at[sidx_v.at[pl.ds(off, sz)]],
                              _dst(sbufs[k], sz), ssems[k]).wait()
        pltpu.make_async_copy(ftab_hbm.at[fidx_v.at[pl.ds(off, sz)]],
                              _dst(fbufs[k], sz), fsems[k]).wait()

    def compute(off, sz, k):
        sbuf, fbuf = sbufs[k], fbufs[k]

        def group(g, carry):
            r0 = g * 16
            lane = lax.iota(jnp.int32, 16)
            acc = jnp.zeros((16,), jnp.float32)
            for j in range(16):
                r = r0 + j
                p = (sbuf[r, pl.ds(0, 16)] * fbuf[r, pl.ds(0, 16)]
                     + sbuf[r, pl.ds(16, 16)] * fbuf[r, pl.ds(16, 16)]
                     + sbuf[r, pl.ds(32, 16)] * fbuf[r, pl.ds(32, 16)]
                     + sbuf[r, pl.ds(48, 16)] * fbuf[r, pl.ds(48, 16)])
                acc = jnp.where(lane == j, jnp.sum(p), acc)
            out_v[pl.ds(off + r0, 16)] = acc
            return carry

        lax.fori_loop(0, sz // 16, group, 0)

    for k in range(NBUF):
        start_tail(k, k)

    M = (UNIFORM - NBUF) // NBUF

    def rotation(i, carry):
        for k in range(NBUF):
            off = (NBUF * i + k) * R
            wait_dyn(off, k)
            compute(off, R, k)
            start_dyn(off + NBUF * R, k)
        return carry

    lax.fori_loop(0, M, rotation, 0)

    for c in range(M * NBUF, NCHUNK):
        wait_tail(c, c % NBUF)
        compute(OFFS[c], SIZES[c], c % NBUF)
        if c + NBUF < NCHUNK:
            start_tail(c + NBUF, c % NBUF)

    pltpu.sync_copy(out_v, out_hbm.at[pl.ds(wid * PER_W, PER_W)])


@jax.jit
def kernel(sample_id, filename, sample_table, filename_table):
    sid = sample_id.reshape(NW, PER_W).astype(jnp.int32)
    fid = filename.reshape(NW, PER_W).astype(jnp.int32)
    mesh = plsc.VectorSubcoreMesh(core_axis_name="c", subcore_axis_name="s")
    run = pl.kernel(
        _sc_body,
        out_type=jax.ShapeDtypeStruct((N,), jnp.float32),
        mesh=mesh,
        scratch_types=[
            pltpu.VMEM((PER_W,), jnp.int32),
            pltpu.VMEM((PER_W,), jnp.int32),
        ] + [pltpu.VMEM((R, D), jnp.float32)] * (2 * NBUF) + [
            pltpu.VMEM((PER_W,), jnp.float32),
        ] + [pltpu.SemaphoreType.DMA] * (2 * NBUF),
        compiler_params=pltpu.CompilerParams(
            needs_layout_passes=False, use_tc_tiling_on_sc=False),
    )
    out = run(sid, fid, sample_table, filename_table)
    return out.reshape(B, H)


# speedup vs baseline: 1.0111x; 1.0111x over previous
import functools

import jax
import jax.numpy as jnp
from jax import lax
from jax.experimental import pallas as pl
from jax.experimental.pallas import tpu as pltpu
from jax.experimental.pallas import tpu_sc as plsc

B = 4096
H = 50
D = 64
N = B * H
NC = 2
NS = 16
NW = NC * NS
PER_W = N // NW
R = 416
NBUF = 2

SIZES = [R] * 15 + [160]
OFFS = [sum(SIZES[:i]) for i in range(len(SIZES))]
NCHUNK = len(SIZES)
assert sum(SIZES) == PER_W
UNIFORM = 15


def _sc_body(sid_hbm, fid_hbm, stab_hbm, ftab_hbm, out_hbm,
             sidx_v, fidx_v, s0, s1, f0, f1, out_v,
             sem_s0, sem_s1, sem_f0, sem_f1):
    wid = lax.axis_index("s") * NC + lax.axis_index("c")

    sbufs = (s0, s1)
    fbufs = (f0, f1)
    ssems = (sem_s0, sem_s1)
    fsems = (sem_f0, sem_f1)

    pltpu.sync_copy(sid_hbm.at[wid], sidx_v)
    pltpu.sync_copy(fid_hbm.at[wid], fidx_v)

    def start_dyn(off, k):
        pltpu.async_copy(stab_hbm.at[sidx_v.at[pl.ds(off, R)]],
                         sbufs[k], ssems[k])
        pltpu.async_copy(ftab_hbm.at[fidx_v.at[pl.ds(off, R)]],
                         fbufs[k], fsems[k])

    def wait_dyn(off, k):
        pltpu.make_async_copy(stab_hbm.at[sidx_v.at[pl.ds(off, R)]],
                              sbufs[k], ssems[k]).wait()
        pltpu.make_async_copy(ftab_hbm.at[fidx_v.at[pl.ds(off, R)]],
                              fbufs[k], fsems[k]).wait()

    def _dst(buf, sz):
        return buf if sz == R else buf.at[pl.ds(0, sz)]

    def start_tail(c, k):
        off, sz = OFFS[c], SIZES[c]
        pltpu.async_copy(stab_hbm.at[sidx_v.at[pl.ds(off, sz)]],
                         _dst(sbufs[k], sz), ssems[k])
        pltpu.async_copy(ftab_hbm.at[fidx_v.at[pl.ds(off, sz)]],
                         _dst(fbufs[k], sz), fsems[k])

    def wait_tail(c, k):
        off, sz = OFFS[c], SIZES[c]
        pltpu.make_async_copy(stab_hbm.at[sidx_v.at[pl.ds(off, sz)]],
                              _dst(sbufs[k], sz), ssems[k]).wait()
        pltpu.make_async_copy(ftab_hbm.at[fidx_v.at[pl.ds(off, sz)]],
                              _dst(fbufs[k], sz), fsems[k]).wait()

    def compute(off, sz, k):
        sbuf, fbuf = sbufs[k], fbufs[k]

        def group(g, carry):
            r0 = g * 16
            lane = lax.iota(jnp.int32, 16)
            acc = jnp.zeros((16,), jnp.float32)
            for j in range(16):
                r = r0 + j
                p = (sbuf[r, pl.ds(0, 16)] * fbuf[r, pl.ds(0, 16)]
                     + sbuf[r, pl.ds(16, 16)] * fbuf[r, pl.ds(16, 16)]
                     + sbuf[r, pl.ds(32, 16)] * fbuf[r, pl.ds(32, 16)]
                     + sbuf[r, pl.ds(48, 16)] * fbuf[r, pl.ds(48, 16)])
                acc = jnp.where(lane == j, jnp.sum(p), acc)
            out_v[pl.ds(off + r0, 16)] = acc
            return carry

        lax.fori_loop(0, sz // 16, group, 0)

    for k in range(NBUF):
        start_tail(k, k)

    M = (UNIFORM - NBUF) // NBUF

    def rotation(i, carry):
        for k in range(NBUF):
            off = (NBUF * i + k) * R
            wait_dyn(off, k)
            compute(off, R, k)
            start_dyn(off + NBUF * R, k)
        return carry

    lax.fori_loop(0, M, rotation, 0)

    for c in range(M * NBUF, NCHUNK):
        wait_tail(c, c % NBUF)
        compute(OFFS[c], SIZES[c], c % NBUF)
        if c + NBUF < NCHUNK:
            start_tail(c + NBUF, c % NBUF)

    pltpu.sync_copy(out_v, out_hbm.at[pl.ds(wid * PER_W, PER_W)])


@jax.jit
def kernel(sample_id, filename, sample_table, filename_table):
    sid = sample_id.reshape(NW, PER_W).astype(jnp.int32)
    fid = filename.reshape(NW, PER_W).astype(jnp.int32)
    mesh = plsc.VectorSubcoreMesh(core_axis_name="c", subcore_axis_name="s")
    run = pl.kernel(
        _sc_body,
        out_type=jax.ShapeDtypeStruct((N,), jnp.float32),
        mesh=mesh,
        scratch_types=[
            pltpu.VMEM((PER_W,), jnp.int32),
            pltpu.VMEM((PER_W,), jnp.int32),
        ] + [pltpu.VMEM((R, D), jnp.float32)] * (2 * NBUF) + [
            pltpu.VMEM((PER_W,), jnp.float32),
        ] + [pltpu.SemaphoreType.DMA] * (2 * NBUF),
        compiler_params=pltpu.CompilerParams(
            needs_layout_passes=False, use_tc_tiling_on_sc=False),
    )
    out = run(sid, fid, sample_table, filename_table)
    return out.reshape(B, H)


# speedup vs baseline: 1.0187x; 1.0075x over previous
import functools

import jax
import jax.numpy as jnp
from jax import lax
from jax.experimental import pallas as pl
from jax.experimental.pallas import tpu as pltpu
from jax.experimental.pallas import tpu_sc as plsc

B = 4096
H = 50
D = 64
N = B * H
NC = 2
NS = 16
NW = NC * NS
PER_W = N // NW
R = 400
NBUF = 2

SIZES = [R] * 16
OFFS = [sum(SIZES[:i]) for i in range(len(SIZES))]
NCHUNK = len(SIZES)
assert sum(SIZES) == PER_W
UNIFORM = 16


def _sc_body(sid_hbm, fid_hbm, stab_hbm, ftab_hbm, out_hbm,
             sidx_v, fidx_v, s0, s1, f0, f1, out_v,
             sem_s0, sem_s1, sem_f0, sem_f1):
    wid = lax.axis_index("s") * NC + lax.axis_index("c")

    sbufs = (s0, s1)
    fbufs = (f0, f1)
    ssems = (sem_s0, sem_s1)
    fsems = (sem_f0, sem_f1)

    pltpu.sync_copy(sid_hbm.at[wid], sidx_v)
    pltpu.sync_copy(fid_hbm.at[wid], fidx_v)

    lane = lax.iota(jnp.int32, 16)
    masks = tuple(lane == j for j in range(16))

    def start_dyn(off, k):
        pltpu.async_copy(stab_hbm.at[sidx_v.at[pl.ds(off, R)]],
                         sbufs[k], ssems[k])
        pltpu.async_copy(ftab_hbm.at[fidx_v.at[pl.ds(off, R)]],
                         fbufs[k], fsems[k])

    def wait_dyn(off, k):
        pltpu.make_async_copy(stab_hbm.at[sidx_v.at[pl.ds(off, R)]],
                              sbufs[k], ssems[k]).wait()
        pltpu.make_async_copy(ftab_hbm.at[fidx_v.at[pl.ds(off, R)]],
                              fbufs[k], fsems[k]).wait()

    def _dst(buf, sz):
        return buf if sz == R else buf.at[pl.ds(0, sz)]

    def start_tail(c, k):
        off, sz = OFFS[c], SIZES[c]
        pltpu.async_copy(stab_hbm.at[sidx_v.at[pl.ds(off, sz)]],
                         _dst(sbufs[k], sz), ssems[k])
        pltpu.async_copy(ftab_hbm.at[fidx_v.at[pl.ds(off, sz)]],
                         _dst(fbufs[k], sz), fsems[k])

    def wait_tail(c, k):
        off, sz = OFFS[c], SIZES[c]
        pltpu.make_async_copy(stab_hbm.at[sidx_v.at[pl.ds(off, sz)]],
                              _dst(sbufs[k], sz), ssems[k]).wait()
        pltpu.make_async_copy(ftab_hbm.at[fidx_v.at[pl.ds(off, sz)]],
                              _dst(fbufs[k], sz), fsems[k]).wait()

    def compute(off, sz, k):
        sbuf, fbuf = sbufs[k], fbufs[k]

        def group(g, carry):
            r0 = g * 16
            acc = jnp.zeros((16,), jnp.float32)
            for j in range(16):
                r = r0 + j
                p = (sbuf[r, pl.ds(0, 16)] * fbuf[r, pl.ds(0, 16)]
                     + sbuf[r, pl.ds(16, 16)] * fbuf[r, pl.ds(16, 16)]
                     + sbuf[r, pl.ds(32, 16)] * fbuf[r, pl.ds(32, 16)]
                     + sbuf[r, pl.ds(48, 16)] * fbuf[r, pl.ds(48, 16)])
                acc = jnp.where(masks[j], jnp.sum(p), acc)
            out_v[pl.ds(off + r0, 16)] = acc
            return carry

        lax.fori_loop(0, sz // 16, group, 0)

    for k in range(NBUF):
        start_tail(k, k)

    M = (UNIFORM - NBUF) // NBUF

    def rotation(i, carry):
        for k in range(NBUF):
            off = (NBUF * i + k) * R
            wait_dyn(off, k)
            compute(off, R, k)
            start_dyn(off + NBUF * R, k)
        return carry

    lax.fori_loop(0, M, rotation, 0)

    for c in range(M * NBUF, NCHUNK):
        wait_tail(c, c % NBUF)
        compute(OFFS[c], SIZES[c], c % NBUF)
        if c + NBUF < NCHUNK:
            start_tail(c + NBUF, c % NBUF)

    pltpu.sync_copy(out_v, out_hbm.at[pl.ds(wid * PER_W, PER_W)])


@jax.jit
def kernel(sample_id, filename, sample_table, filename_table):
    sid = sample_id.reshape(NW, PER_W).astype(jnp.int32)
    fid = filename.reshape(NW, PER_W).astype(jnp.int32)
    mesh = plsc.VectorSubcoreMesh(core_axis_name="c", subcore_axis_name="s")
    run = pl.kernel(
        _sc_body,
        out_type=jax.ShapeDtypeStruct((N,), jnp.float32),
        mesh=mesh,
        scratch_types=[
            pltpu.VMEM((PER_W,), jnp.int32),
            pltpu.VMEM((PER_W,), jnp.int32),
        ] + [pltpu.VMEM((R, D), jnp.float32)] * (2 * NBUF) + [
            pltpu.VMEM((PER_W,), jnp.float32),
        ] + [pltpu.SemaphoreType.DMA] * (2 * NBUF),
        compiler_params=pltpu.CompilerParams(
            needs_layout_passes=False, use_tc_tiling_on_sc=False),
    )
    out = run(sid, fid, sample_table, filename_table)
    return out.reshape(B, H)


# speedup vs baseline: 1.0201x; 1.0013x over previous
import jax
import jax.numpy as jnp
from jax import lax
from jax.experimental import pallas as pl
from jax.experimental.pallas import tpu as pltpu
from jax.experimental.pallas import tpu_sc as plsc

B = 4096
H = 50
D = 64
N = B * H
NC = 2
NS = 16
NW = NC * NS
PER_W = N // NW
R = 400
NBUF = 2

SIZES = [R] * 16
OFFS = [sum(SIZES[:i]) for i in range(len(SIZES))]
NCHUNK = len(SIZES)
assert sum(SIZES) == PER_W
UNIFORM = 16


def _sc_body(sid_hbm, fid_hbm, stab_hbm, ftab_hbm, out_hbm,
             sidx_v, fidx_v, s0, s1, f0, f1, out_v,
             sem_s0, sem_s1, sem_f0, sem_f1):
    wid = lax.axis_index("s") * NC + lax.axis_index("c")

    sbufs = (s0, s1)
    fbufs = (f0, f1)
    ssems = (sem_s0, sem_s1)
    fsems = (sem_f0, sem_f1)

    pltpu.sync_copy(sid_hbm.at[wid], sidx_v)
    pltpu.sync_copy(fid_hbm.at[wid], fidx_v)

    lane = lax.iota(jnp.int32, 16)
    masks = tuple(lane == j for j in range(16))

    def start_dyn(off, k):
        pltpu.async_copy(stab_hbm.at[sidx_v.at[pl.ds(off, R)]],
                         sbufs[k], ssems[k])
        pltpu.async_copy(ftab_hbm.at[fidx_v.at[pl.ds(off, R)]],
                         fbufs[k], fsems[k])

    def wait_dyn(off, k):
        pltpu.make_async_copy(stab_hbm.at[sidx_v.at[pl.ds(off, R)]],
                              sbufs[k], ssems[k]).wait()
        pltpu.make_async_copy(ftab_hbm.at[fidx_v.at[pl.ds(off, R)]],
                              fbufs[k], fsems[k]).wait()

    def _dst(buf, sz):
        return buf if sz == R else buf.at[pl.ds(0, sz)]

    def start_tail(c, k):
        off, sz = OFFS[c], SIZES[c]
        pltpu.async_copy(stab_hbm.at[sidx_v.at[pl.ds(off, sz)]],
                         _dst(sbufs[k], sz), ssems[k])
        pltpu.async_copy(ftab_hbm.at[fidx_v.at[pl.ds(off, sz)]],
                         _dst(fbufs[k], sz), fsems[k])

    def wait_tail(c, k):
        off, sz = OFFS[c], SIZES[c]
        pltpu.make_async_copy(stab_hbm.at[sidx_v.at[pl.ds(off, sz)]],
                              _dst(sbufs[k], sz), ssems[k]).wait()
        pltpu.make_async_copy(ftab_hbm.at[fidx_v.at[pl.ds(off, sz)]],
                              _dst(fbufs[k], sz), fsems[k]).wait()

    def compute(off, sz, k):
        sbuf, fbuf = sbufs[k], fbufs[k]

        def group(g, carry):
            r0 = g * 16
            acc = jnp.zeros((16,), jnp.float32)
            for j in range(16):
                r = r0 + j
                p = (sbuf[r, pl.ds(0, 16)] * fbuf[r, pl.ds(0, 16)]
                     + sbuf[r, pl.ds(16, 16)] * fbuf[r, pl.ds(16, 16)]
                     + sbuf[r, pl.ds(32, 16)] * fbuf[r, pl.ds(32, 16)]
                     + sbuf[r, pl.ds(48, 16)] * fbuf[r, pl.ds(48, 16)])
                acc = jnp.where(masks[j], jnp.sum(p), acc)
            out_v[pl.ds(off + r0, 16)] = acc
            return carry

        lax.fori_loop(0, sz // 16, group, 0)

    for k in range(NBUF):
        start_tail(k, k)

    M = (UNIFORM - NBUF) // NBUF

    def rotation(i, carry):
        for k in range(NBUF):
            off = (NBUF * i + k) * R
            wait_dyn(off, k)
            compute(off, R, k)
            start_dyn(off + NBUF * R, k)
        return carry

    lax.fori_loop(0, M, rotation, 0)

    for c in range(M * NBUF, NCHUNK):
        wait_tail(c, c % NBUF)
        compute(OFFS[c], SIZES[c], c % NBUF)
        if c + NBUF < NCHUNK:
            start_tail(c + NBUF, c % NBUF)

    pltpu.sync_copy(out_v, out_hbm.at[pl.ds(wid * PER_W, PER_W)])


@jax.jit
def kernel(sample_id, filename, sample_table, filename_table):
    sid = sample_id.reshape(NW, PER_W).astype(jnp.int32)
    fid = filename.reshape(NW, PER_W).astype(jnp.int32)
    mesh = plsc.VectorSubcoreMesh(core_axis_name="c", subcore_axis_name="s")
    run = pl.kernel(
        _sc_body,
        out_type=jax.ShapeDtypeStruct((N,), jnp.float32),
        mesh=mesh,
        scratch_types=[
            pltpu.VMEM((PER_W,), jnp.int32),
            pltpu.VMEM((PER_W,), jnp.int32),
        ] + [pltpu.VMEM((R, D), jnp.float32)] * (2 * NBUF) + [
            pltpu.VMEM((PER_W,), jnp.float32),
        ] + [pltpu.SemaphoreType.DMA] * (2 * NBUF),
        compiler_params=pltpu.CompilerParams(
            needs_layout_passes=False, use_tc_tiling_on_sc=False),
    )
    out = run(sid, fid, sample_table, filename_table)
    return out.reshape(B, H)
